# baseline (speedup 1.0000x reference)
import functools
import jax
import jax.numpy as jnp
from jax import lax
from jax.experimental import pallas as pl
from jax.experimental.pallas import tpu as pltpu
from jax.experimental.pallas import tpu_sc as plsc

HIDDEN = 1024
FFN = 3584
NUM_EXPERTS = 8
TOP_K = 2
T = 2048
NSLOT = T * TOP_K

BT = 128
NBMAX = NSLOT // BT + NUM_EXPERTS
SP = NBMAX * BT
FB = 1792
NF = FFN // FB

NC = 2
NS = 16
NW = NC * NS


def _router_body(x_ref, gate_ref, ei_ref, wn_ref):
    x = x_ref[...]
    logits = lax.dot_general(x, gate_ref[...], (((1,), (1,)), ((), ())),
                             preferred_element_type=jnp.float32)
    m = jnp.max(logits, axis=-1, keepdims=True)
    p = jnp.exp(logits - m)
    rw = p / jnp.sum(p, axis=-1, keepdims=True)
    lane = lax.broadcasted_iota(jnp.int32, rw.shape, 1)
    m1 = jnp.max(rw, axis=-1, keepdims=True)
    i1 = jnp.min(jnp.where(rw == m1, lane, NUM_EXPERTS), axis=-1, keepdims=True)
    rw2 = jnp.where(lane == i1, -jnp.inf, rw)
    m2 = jnp.max(rw2, axis=-1, keepdims=True)
    i2 = jnp.min(jnp.where(rw2 == m2, lane, NUM_EXPERTS), axis=-1, keepdims=True)
    s = m1 + m2
    ei_ref[...] = jnp.concatenate([i1, i2], axis=1)
    wn_ref[...] = jnp.concatenate([m1 / s, m2 / s], axis=1)


def _router(x, gate_w):
    return pl.pallas_call(
        _router_body,
        out_shape=[
            jax.ShapeDtypeStruct((T, TOP_K), jnp.int32),
            jax.ShapeDtypeStruct((T, TOP_K), jnp.float32),
        ],
    )(x, gate_w)


def _bookkeeping(ei, wn):
    e_slot = ei.reshape(-1)
    w_slot = wn.reshape(-1)
    order = jnp.argsort(e_slot, stable=True)
    counts = jnp.zeros((NUM_EXPERTS,), jnp.int32).at[e_slot].add(1)
    blocks = (counts + BT - 1) // BT
    cumblocks = jnp.cumsum(blocks)
    base = BT * (cumblocks - blocks)
    gstart = jnp.cumsum(counts) - counts
    e_j = e_slot[order]
    p_j = base[e_j] + (jnp.arange(NSLOT, dtype=jnp.int32) - gstart[e_j])
    ws = jnp.zeros((SP,), jnp.float32).at[p_j].set(w_slot[order])
    inv = jnp.zeros((NSLOT,), jnp.int32).at[order].set(p_j)
    inv2 = inv.reshape(T, TOP_K)
    inv0 = inv2[:, 0] + 0
    inv1 = inv2[:, 1] + 0
    bexp = jnp.minimum(
        jnp.searchsorted(cumblocks, jnp.arange(NBMAX, dtype=jnp.int32),
                         side="right"),
        NUM_EXPERTS - 1).astype(jnp.int32)
    meta = jnp.full((8,), cumblocks[-1], jnp.int32)
    return ws, inv, inv0, inv1, bexp, meta


_TOKW = T // NW


def _xs_scatter_body(x_hbm, inv0_hbm, inv1_hbm, xs_hbm, p0_v, p1_v, xrows,
                     gsem, s0, s1):
    wid = lax.axis_index("s") * NC + lax.axis_index("c")
    tbase = wid * _TOKW
    ld = pltpu.async_copy(x_hbm.at[pl.ds(tbase, _TOKW)], xrows, gsem)
    pltpu.sync_copy(inv0_hbm.at[pl.ds(tbase, _TOKW)], p0_v)
    pltpu.sync_copy(inv1_hbm.at[pl.ds(tbase, _TOKW)], p1_v)
    ld.wait()
    st0 = pltpu.async_copy(xrows, xs_hbm.at[p0_v], s0)
    st1 = pltpu.async_copy(xrows, xs_hbm.at[p1_v], s1)
    st0.wait()
    st1.wait()


@functools.cache
def _make_xs_scatter():
    return pl.kernel(
        _xs_scatter_body,
        out_type=jax.ShapeDtypeStruct((SP, HIDDEN), jnp.float32),
        mesh=plsc.VectorSubcoreMesh(core_axis_name="c", subcore_axis_name="s",
                                    num_cores=NC, num_subcores=NS),
        scratch_types=[
            pltpu.VMEM((_TOKW,), jnp.int32),
            pltpu.VMEM((_TOKW,), jnp.int32),
            pltpu.VMEM((_TOKW, HIDDEN), jnp.float32),
            pltpu.SemaphoreType.DMA,
            pltpu.SemaphoreType.DMA,
            pltpu.SemaphoreType.DMA,
        ],
    )


def _xs_scatter(x, inv0, inv1):
    return _make_xs_scatter()(x, inv0, inv1)


def _gmm_body(bexp, meta, xs_ref, w1_ref, w3_ref, w2_ref, ws_ref, yin_ref,
              ys_ref):
    f = pl.program_id(0)
    b = pl.program_id(1)
    nused = meta[0]

    @pl.when(b < nused)
    def _():
        x = xs_ref[...]
        h1 = lax.dot_general(x, w1_ref[0], (((1,), (1,)), ((), ())),
                             preferred_element_type=jnp.float32)
        h3 = lax.dot_general(x, w3_ref[0], (((1,), (1,)), ((), ())),
                             preferred_element_type=jnp.float32)
        act = h1 * (1.0 / (1.0 + jnp.exp(-h1))) * h3
        y = lax.dot_general(act, w2_ref[0], (((1,), (1,)), ((), ())),
                            preferred_element_type=jnp.float32)

        @pl.when(f == 0)
        def _():
            ys_ref[...] = y

        @pl.when((f > 0) & (f < NF - 1))
        def _():
            ys_ref[...] = yin_ref[...] + y

        @pl.when(f == NF - 1)
        def _():
            ys_ref[...] = (yin_ref[...] + y) * ws_ref[...]

    @pl.when(b >= nused)
    def _():
        ys_ref[...] = yin_ref[...]


def _gmm(xs, w1, w3, w2, ws2d, bexp, meta):
    grid_spec = pltpu.PrefetchScalarGridSpec(
        num_scalar_prefetch=2,
        grid=(NF, NBMAX),
        in_specs=[
            pl.BlockSpec((BT, HIDDEN), lambda f, b, be, mt: (b, 0)),
            pl.BlockSpec((1, FB, HIDDEN), lambda f, b, be, mt: (be[b], f, 0)),
            pl.BlockSpec((1, FB, HIDDEN), lambda f, b, be, mt: (be[b], f, 0)),
            pl.BlockSpec((1, HIDDEN, FB), lambda f, b, be, mt: (be[b], 0, f)),
            pl.BlockSpec((BT, 1), lambda f, b, be, mt: (b, 0)),
            pl.BlockSpec((BT, HIDDEN),
                         lambda f, b, be, mt: (jnp.where(f == 0, NBMAX - 1, b), 0)),
        ],
        out_specs=pl.BlockSpec((BT, HIDDEN), lambda f, b, be, mt: (b, 0)),
    )
    yin = jnp.zeros((SP, HIDDEN), jnp.float32)
    return pl.pallas_call(
        _gmm_body,
        grid_spec=grid_spec,
        out_shape=jax.ShapeDtypeStruct((SP, HIDDEN), jnp.float32),
        input_output_aliases={7: 0},
        compiler_params=pltpu.CompilerParams(
            dimension_semantics=("arbitrary", "arbitrary"),
        ),
    )(bexp, meta, xs, w1, w3, w2, ws2d, yin)


_TPW = T // NW
_TCH = 16
_NCC = _TPW // _TCH


def _combine_body(ys_hbm, inv_hbm, out_hbm, inv_v, p0, p1, a0, a1,
                  g0, g1, s0, s1):
    wid = lax.axis_index("s") * NC + lax.axis_index("c")
    tbase = wid * _TPW
    pairs = (p0, p1)
    accs = (a0, a1)
    gsems = (g0, g1)
    ssems = (s0, s1)
    pltpu.sync_copy(inv_hbm.at[pl.ds(TOP_K * tbase, TOP_K * _TPW)], inv_v)
    gathers = [None] * _NCC
    stores = [None] * _NCC
    for c in range(2):
        gathers[c] = pltpu.async_copy(
            ys_hbm.at[inv_v.at[pl.ds(c * TOP_K * _TCH, TOP_K * _TCH)]],
            pairs[c], gsems[c])
    for c in range(_NCC):
        gathers[c].wait()
        if c >= 2:
            stores[c - 2].wait()
        pair_v = pairs[c % 2]
        acc_v = accs[c % 2]

        def add_body(i, carry):
            r = i // (HIDDEN // 64)
            q = (i % (HIDDEN // 64)) * 64
            for u in range(4):
                acc_v[r, pl.ds(q + u * 16, 16)] = (
                    pair_v[2 * r, pl.ds(q + u * 16, 16)]
                    + pair_v[2 * r + 1, pl.ds(q + u * 16, 16)])
            return carry

        lax.fori_loop(0, _TCH * (HIDDEN // 64), add_body, 0)
        stores[c] = pltpu.async_copy(
            acc_v, out_hbm.at[pl.ds(tbase + c * _TCH, _TCH)], ssems[c % 2])
        if c + 2 < _NCC:
            gathers[c + 2] = pltpu.async_copy(
                ys_hbm.at[inv_v.at[pl.ds((c + 2) * TOP_K * _TCH,
                                         TOP_K * _TCH)]],
                pairs[c % 2], gsems[c % 2])
    stores[_NCC - 2].wait()
    stores[_NCC - 1].wait()


@functools.cache
def _make_combine():
    return pl.kernel(
        _combine_body,
        out_type=jax.ShapeDtypeStruct((T, HIDDEN), jnp.float32),
        mesh=plsc.VectorSubcoreMesh(core_axis_name="c", subcore_axis_name="s",
                                    num_cores=NC, num_subcores=NS),
        scratch_types=[
            pltpu.VMEM((TOP_K * _TPW,), jnp.int32),
            pltpu.VMEM((TOP_K * _TCH, HIDDEN), jnp.float32),
            pltpu.VMEM((TOP_K * _TCH, HIDDEN), jnp.float32),
            pltpu.VMEM((_TCH, HIDDEN), jnp.float32),
            pltpu.VMEM((_TCH, HIDDEN), jnp.float32),
            pltpu.SemaphoreType.DMA,
            pltpu.SemaphoreType.DMA,
            pltpu.SemaphoreType.DMA,
            pltpu.SemaphoreType.DMA,
        ],
    )


def _combine(ys, inv):
    return _make_combine()(ys, inv)


@jax.jit
def _moe(x, gate_w, w1, w2, w3):
    ei, wn = _router(x, gate_w)
    ws, inv, inv0, inv1, bexp, meta = _bookkeeping(ei, wn)
    xs = _xs_scatter(x, inv0, inv1)
    ys = _gmm(xs, w1, w3, w2, ws.reshape(SP, 1), bexp, meta)
    return _combine(ys, inv)


def kernel(hidden_states, gate_w, w1, w2, w3):
    B, S, H = hidden_states.shape
    x = hidden_states.reshape(-1, H)
    out = _moe(x, gate_w, w1, w2, w3)
    return out.reshape(B, S, H)


# speedup vs baseline: 1.5802x; 1.5802x over previous
import functools
import jax
import jax.numpy as jnp
from jax import lax
from jax.experimental import pallas as pl
from jax.experimental.pallas import tpu as pltpu
from jax.experimental.pallas import tpu_sc as plsc

HIDDEN = 1024
FFN = 3584
NUM_EXPERTS = 8
TOP_K = 2
T = 2048
NSLOT = T * TOP_K

BT = 256
NBMAX = NSLOT // BT + NUM_EXPERTS
SP = NBMAX * BT
FB = 1792
NF = FFN // FB

NC = 2
NS = 16
NW = NC * NS

_CB = 256


def _router_body(x_ref, gate_ref, inv0_ref, inv1_ref, w0_ref, w1_ref,
                 bexp_ref, gend_ref, meta_ref):
    x = x_ref[...]
    logits = lax.dot_general(x, gate_ref[...], (((1,), (1,)), ((), ())),
                             preferred_element_type=jnp.float32)
    m = jnp.max(logits, axis=-1, keepdims=True)
    p = jnp.exp(logits - m)
    rw = p / jnp.sum(p, axis=-1, keepdims=True)
    lane = lax.broadcasted_iota(jnp.int32, rw.shape, 1)
    m1 = jnp.max(rw, axis=-1, keepdims=True)
    i1 = jnp.min(jnp.where(rw == m1, lane, NUM_EXPERTS), axis=-1,
                 keepdims=True)
    rw2 = jnp.where(lane == i1, -jnp.inf, rw)
    m2 = jnp.max(rw2, axis=-1, keepdims=True)
    i2 = jnp.min(jnp.where(rw2 == m2, lane, NUM_EXPERTS), axis=-1,
                 keepdims=True)
    s = m1 + m2
    w0_ref[...] = m1 / s
    w1_ref[...] = m2 / s

    oh0 = (lane == i1).astype(jnp.float32)
    oh1 = (lane == i2).astype(jnp.float32)
    ohsum = oh0 + oh1

    ri = lax.broadcasted_iota(jnp.int32, (_CB, _CB), 0)
    ci = lax.broadcasted_iota(jnp.int32, (_CB, _CB), 1)
    tril_strict = (ci < ri).astype(jnp.float32)
    cntb_parts = []
    carry = jnp.zeros((1, NUM_EXPERTS), jnp.float32)
    for c in range(T // _CB):
        blk = ohsum[c * _CB:(c + 1) * _CB, :]
        pref = lax.dot_general(tril_strict, blk, (((1,), (0,)), ((), ())),
                               preferred_element_type=jnp.float32)
        cntb_parts.append(pref + carry)
        carry = carry + jnp.sum(blk, axis=0, keepdims=True)
    cntb = jnp.concatenate(cntb_parts, axis=0)
    counts = carry

    counts_i = counts.astype(jnp.int32)
    blocks = (counts_i + BT - 1) // BT
    blocks_f = blocks.astype(jnp.float32)
    ei = lax.broadcasted_iota(jnp.int32, (NUM_EXPERTS, NUM_EXPERTS), 0)
    ej = lax.broadcasted_iota(jnp.int32, (NUM_EXPERTS, NUM_EXPERTS), 1)
    lincl = (ej <= ei).astype(jnp.float32)
    eye = (ej == ei).astype(jnp.float32)
    cumblocks_col = lax.dot_general(
        lincl, blocks_f, (((1,), (1,)), ((), ())),
        preferred_element_type=jnp.float32).astype(jnp.int32)
    blocks_col = lax.dot_general(
        eye, blocks_f, (((1,), (1,)), ((), ())),
        preferred_element_type=jnp.float32).astype(jnp.int32)
    counts_col = lax.dot_general(
        eye, counts, (((1,), (1,)), ((), ())),
        preferred_element_type=jnp.float32).astype(jnp.int32)
    base_col = BT * (cumblocks_col - blocks_col)
    nused = jnp.sum(blocks, axis=-1, keepdims=True)

    basesel0 = lax.dot_general(oh0, base_col.astype(jnp.float32),
                               (((1,), (0,)), ((), ())),
                               preferred_element_type=jnp.float32)
    basesel1 = lax.dot_general(oh1, base_col.astype(jnp.float32),
                               (((1,), (0,)), ((), ())),
                               preferred_element_type=jnp.float32)
    rank0 = jnp.sum(oh0 * cntb, axis=-1, keepdims=True)
    rank1 = jnp.sum(oh1 * cntb, axis=-1, keepdims=True)
    inv0_ref[...] = (basesel0 + rank0).astype(jnp.int32)
    inv1_ref[...] = (basesel1 + rank1).astype(jnp.int32)

    b_lane = lax.broadcasted_iota(jnp.int32, (NUM_EXPERTS, NBMAX), 1)
    ones_b = jnp.ones((1, NBMAX), jnp.int32)
    cum_b = cumblocks_col * ones_b
    blk_b = blocks_col * ones_b
    bexp = jnp.sum((cum_b <= b_lane).astype(jnp.int32), axis=0,
                   keepdims=True)
    bexp_ref[...] = jnp.minimum(bexp, NUM_EXPERTS - 1)
    ind = ((cum_b > b_lane) & (cum_b - blk_b <= b_lane)).astype(jnp.int32)
    gend_col = base_col + counts_col
    gend_ref[...] = jnp.sum(ind * (gend_col * ones_b), axis=0, keepdims=True)
    meta_ref[...] = nused * jnp.ones((1, 8), jnp.int32)


def _router(x, gate_w):
    return pl.pallas_call(
        _router_body,
        out_shape=[
            jax.ShapeDtypeStruct((T, 1), jnp.int32),
            jax.ShapeDtypeStruct((T, 1), jnp.int32),
            jax.ShapeDtypeStruct((T, 1), jnp.float32),
            jax.ShapeDtypeStruct((T, 1), jnp.float32),
            jax.ShapeDtypeStruct((1, NBMAX), jnp.int32),
            jax.ShapeDtypeStruct((1, NBMAX), jnp.int32),
            jax.ShapeDtypeStruct((1, 8), jnp.int32),
        ],
    )(x, gate_w)


_TOKW = T // NW


def _xs_scatter_body(x_hbm, inv0_hbm, inv1_hbm, w0_hbm, w1_hbm,
                     xs_hbm, ws_hbm, p0_v, p1_v, w0_v, w1_v, xrows,
                     gsem, s0, s1, s2, s3):
    wid = lax.axis_index("s") * NC + lax.axis_index("c")
    tbase = wid * _TOKW
    ld = pltpu.async_copy(x_hbm.at[pl.ds(tbase, _TOKW)], xrows, gsem)
    pltpu.sync_copy(inv0_hbm.at[pl.ds(tbase, _TOKW)], p0_v)
    pltpu.sync_copy(inv1_hbm.at[pl.ds(tbase, _TOKW)], p1_v)
    pltpu.sync_copy(w0_hbm.at[pl.ds(tbase, _TOKW)], w0_v)
    pltpu.sync_copy(w1_hbm.at[pl.ds(tbase, _TOKW)], w1_v)
    st2 = pltpu.async_copy(w0_v, ws_hbm.at[p0_v], s2)
    st3 = pltpu.async_copy(w1_v, ws_hbm.at[p1_v], s3)
    ld.wait()
    st0 = pltpu.async_copy(xrows, xs_hbm.at[p0_v], s0)
    st1 = pltpu.async_copy(xrows, xs_hbm.at[p1_v], s1)
    st0.wait()
    st1.wait()
    st2.wait()
    st3.wait()


@functools.cache
def _make_xs_scatter():
    return pl.kernel(
        _xs_scatter_body,
        out_type=[
            jax.ShapeDtypeStruct((SP, HIDDEN), jnp.float32),
            jax.ShapeDtypeStruct((SP,), jnp.float32),
        ],
        mesh=plsc.VectorSubcoreMesh(core_axis_name="c", subcore_axis_name="s",
                                    num_cores=NC, num_subcores=NS),
        scratch_types=[
            pltpu.VMEM((_TOKW,), jnp.int32),
            pltpu.VMEM((_TOKW,), jnp.int32),
            pltpu.VMEM((_TOKW,), jnp.float32),
            pltpu.VMEM((_TOKW,), jnp.float32),
            pltpu.VMEM((_TOKW, HIDDEN), jnp.float32),
            pltpu.SemaphoreType.DMA,
            pltpu.SemaphoreType.DMA,
            pltpu.SemaphoreType.DMA,
            pltpu.SemaphoreType.DMA,
            pltpu.SemaphoreType.DMA,
        ],
    )


def _xs_scatter(x, inv0, inv1, w0, w1):
    return _make_xs_scatter()(x, inv0, inv1, w0, w1)


def _gmm_body(bexp, meta, gend, xs_ref, w1_ref, w3_ref, w2_ref, ws_ref,
              yin_ref, ys_ref):
    f = pl.program_id(0)
    b = pl.program_id(1)
    nused = meta[0]

    @pl.when(b < nused)
    def _():
        x = xs_ref[...]
        h1 = lax.dot_general(x, w1_ref[0], (((1,), (1,)), ((), ())),
                             preferred_element_type=jnp.float32)
        h3 = lax.dot_general(x, w3_ref[0], (((1,), (1,)), ((), ())),
                             preferred_element_type=jnp.float32)
        act = h1 * (1.0 / (1.0 + jnp.exp(-h1))) * h3
        y = lax.dot_general(act, w2_ref[0], (((1,), (1,)), ((), ())),
                            preferred_element_type=jnp.float32)

        @pl.when(f == 0)
        def _():
            ys_ref[...] = y

        @pl.when((f > 0) & (f < NF - 1))
        def _():
            ys_ref[...] = yin_ref[...] + y

        @pl.when(f == NF - 1)
        def _():
            rows = b * BT + lax.broadcasted_iota(jnp.int32, (BT, 1), 0)
            wsv = jnp.where(rows < gend[b], ws_ref[...], 0.0)
            ys_ref[...] = (yin_ref[...] + y) * wsv

    @pl.when(b >= nused)
    def _():
        ys_ref[...] = yin_ref[...]


def _gmm(xs, w1, w3, w2, ws2d, bexp, meta, gend):
    grid_spec = pltpu.PrefetchScalarGridSpec(
        num_scalar_prefetch=3,
        grid=(NF, NBMAX),
        in_specs=[
            pl.BlockSpec((BT, HIDDEN), lambda f, b, be, mt, ge: (b, 0)),
            pl.BlockSpec((1, FB, HIDDEN),
                         lambda f, b, be, mt, ge: (be[b], f, 0)),
            pl.BlockSpec((1, FB, HIDDEN),
                         lambda f, b, be, mt, ge: (be[b], f, 0)),
            pl.BlockSpec((1, HIDDEN, FB),
                         lambda f, b, be, mt, ge: (be[b], 0, f)),
            pl.BlockSpec((BT, 1), lambda f, b, be, mt, ge: (b, 0)),
            pl.BlockSpec((BT, HIDDEN),
                         lambda f, b, be, mt, ge:
                         (jnp.where(f == 0, NBMAX - 1, b), 0)),
        ],
        out_specs=pl.BlockSpec((BT, HIDDEN), lambda f, b, be, mt, ge: (b, 0)),
    )
    yin = jnp.zeros((SP, HIDDEN), jnp.float32)
    return pl.pallas_call(
        _gmm_body,
        grid_spec=grid_spec,
        out_shape=jax.ShapeDtypeStruct((SP, HIDDEN), jnp.float32),
        input_output_aliases={8: 0},
        compiler_params=pltpu.CompilerParams(
            dimension_semantics=("arbitrary", "arbitrary"),
        ),
    )(bexp, meta, gend, xs, w1, w3, w2, ws2d, yin)


_TPW = T // NW
_TCH = 16
_NCC = _TPW // _TCH


def _combine_body(ys_hbm, inv0_hbm, inv1_hbm, out_hbm, p0_v, p1_v,
                  b0a, b0b, b1a, b1b, acc0, acc1,
                  g0, g1, g2, g3, s0, s1):
    wid = lax.axis_index("s") * NC + lax.axis_index("c")
    tbase = wid * _TPW
    pltpu.sync_copy(inv0_hbm.at[pl.ds(tbase, _TPW)], p0_v)
    pltpu.sync_copy(inv1_hbm.at[pl.ds(tbase, _TPW)], p1_v)
    bufs0 = (b0a, b0b)
    bufs1 = (b1a, b1b)
    accs = (acc0, acc1)
    gsem0 = (g0, g1)
    gsem1 = (g2, g3)
    ssems = (s0, s1)
    gath0 = [None] * _NCC
    gath1 = [None] * _NCC
    stores = [None] * _NCC
    for c in range(2):
        gath0[c] = pltpu.async_copy(
            ys_hbm.at[p0_v.at[pl.ds(c * _TCH, _TCH)]], bufs0[c], gsem0[c])
        gath1[c] = pltpu.async_copy(
            ys_hbm.at[p1_v.at[pl.ds(c * _TCH, _TCH)]], bufs1[c], gsem1[c])
    for c in range(_NCC):
        gath0[c].wait()
        gath1[c].wait()
        if c >= 2:
            stores[c - 2].wait()
        u = c % 2
        pa = bufs0[u]
        pb = bufs1[u]
        acc_v = accs[u]

        def add_body(i, carry):
            r = i // (HIDDEN // 64)
            q = (i % (HIDDEN // 64)) * 64
            for k in range(4):
                acc_v[r, pl.ds(q + k * 16, 16)] = (
                    pa[r, pl.ds(q + k * 16, 16)]
                    + pb[r, pl.ds(q + k * 16, 16)])
            return carry

        lax.fori_loop(0, _TCH * (HIDDEN // 64), add_body, 0)
        stores[c] = pltpu.async_copy(
            acc_v, out_hbm.at[pl.ds(tbase + c * _TCH, _TCH)], ssems[u])
        if c + 2 < _NCC:
            gath0[c + 2] = pltpu.async_copy(
                ys_hbm.at[p0_v.at[pl.ds((c + 2) * _TCH, _TCH)]],
                bufs0[u], gsem0[u])
            gath1[c + 2] = pltpu.async_copy(
                ys_hbm.at[p1_v.at[pl.ds((c + 2) * _TCH, _TCH)]],
                bufs1[u], gsem1[u])
    stores[_NCC - 2].wait()
    stores[_NCC - 1].wait()


@functools.cache
def _make_combine():
    return pl.kernel(
        _combine_body,
        out_type=jax.ShapeDtypeStruct((T, HIDDEN), jnp.float32),
        mesh=plsc.VectorSubcoreMesh(core_axis_name="c", subcore_axis_name="s",
                                    num_cores=NC, num_subcores=NS),
        scratch_types=[
            pltpu.VMEM((_TPW,), jnp.int32),
            pltpu.VMEM((_TPW,), jnp.int32),
            pltpu.VMEM((_TCH, HIDDEN), jnp.float32),
            pltpu.VMEM((_TCH, HIDDEN), jnp.float32),
            pltpu.VMEM((_TCH, HIDDEN), jnp.float32),
            pltpu.VMEM((_TCH, HIDDEN), jnp.float32),
            pltpu.VMEM((_TCH, HIDDEN), jnp.float32),
            pltpu.VMEM((_TCH, HIDDEN), jnp.float32),
            pltpu.SemaphoreType.DMA,
            pltpu.SemaphoreType.DMA,
            pltpu.SemaphoreType.DMA,
            pltpu.SemaphoreType.DMA,
            pltpu.SemaphoreType.DMA,
            pltpu.SemaphoreType.DMA,
        ],
    )


def _combine(ys, inv0, inv1):
    return _make_combine()(ys, inv0, inv1)


@jax.jit
def _moe(x, gate_w, w1, w2, w3):
    inv0, inv1, w0, wv1, bexp, gend, meta = _router(x, gate_w)
    inv0 = inv0.reshape(T)
    inv1 = inv1.reshape(T)
    xs, ws = _xs_scatter(x, inv0, inv1, w0.reshape(T), wv1.reshape(T))
    ys = _gmm(xs, w1, w3, w2, ws.reshape(SP, 1),
              bexp.reshape(NBMAX), meta.reshape(8), gend.reshape(NBMAX))
    return _combine(ys, inv0, inv1)


def kernel(hidden_states, gate_w, w1, w2, w3):
    B, S, H = hidden_states.shape
    x = hidden_states.reshape(-1, H)
    out = _moe(x, gate_w, w1, w2, w3)
    return out.reshape(B, S, H)


# speedup vs baseline: 1.6257x; 1.0288x over previous
import functools
import jax
import jax.numpy as jnp
from jax import lax
from jax.experimental import pallas as pl
from jax.experimental.pallas import tpu as pltpu
from jax.experimental.pallas import tpu_sc as plsc

HIDDEN = 1024
FFN = 3584
NUM_EXPERTS = 8
TOP_K = 2
T = 2048
NSLOT = T * TOP_K

BT = 256
NBMAX = NSLOT // BT + NUM_EXPERTS
SP = NBMAX * BT
FB = 1792
NF = FFN // FB

NC = 2
NS = 16
NW = NC * NS

_CB = 256


def _router_body(x_ref, gate_ref, inv0_ref, inv1_ref, w0_ref, w1_ref,
                 bexp_ref, gend_ref, meta_ref):
    x = x_ref[...]
    logits = lax.dot_general(x, gate_ref[...], (((1,), (1,)), ((), ())),
                             preferred_element_type=jnp.float32)
    m = jnp.max(logits, axis=-1, keepdims=True)
    p = jnp.exp(logits - m)
    rw = p / jnp.sum(p, axis=-1, keepdims=True)
    lane = lax.broadcasted_iota(jnp.int32, rw.shape, 1)
    m1 = jnp.max(rw, axis=-1, keepdims=True)
    i1 = jnp.min(jnp.where(rw == m1, lane, NUM_EXPERTS), axis=-1,
                 keepdims=True)
    rw2 = jnp.where(lane == i1, -jnp.inf, rw)
    m2 = jnp.max(rw2, axis=-1, keepdims=True)
    i2 = jnp.min(jnp.where(rw2 == m2, lane, NUM_EXPERTS), axis=-1,
                 keepdims=True)
    s = m1 + m2
    w0_ref[...] = m1 / s
    w1_ref[...] = m2 / s

    oh0 = (lane == i1).astype(jnp.float32)
    oh1 = (lane == i2).astype(jnp.float32)
    ohsum = oh0 + oh1

    ri = lax.broadcasted_iota(jnp.int32, (_CB, _CB), 0)
    ci = lax.broadcasted_iota(jnp.int32, (_CB, _CB), 1)
    tril_strict = (ci < ri).astype(jnp.float32)
    cntb_parts = []
    carry = jnp.zeros((1, NUM_EXPERTS), jnp.float32)
    for c in range(T // _CB):
        blk = ohsum[c * _CB:(c + 1) * _CB, :]
        pref = lax.dot_general(tril_strict, blk, (((1,), (0,)), ((), ())),
                               preferred_element_type=jnp.float32)
        cntb_parts.append(pref + carry)
        carry = carry + jnp.sum(blk, axis=0, keepdims=True)
    cntb = jnp.concatenate(cntb_parts, axis=0)
    counts = carry

    counts_i = counts.astype(jnp.int32)
    blocks = (counts_i + BT - 1) // BT
    blocks_f = blocks.astype(jnp.float32)
    ei = lax.broadcasted_iota(jnp.int32, (NUM_EXPERTS, NUM_EXPERTS), 0)
    ej = lax.broadcasted_iota(jnp.int32, (NUM_EXPERTS, NUM_EXPERTS), 1)
    lincl = (ej <= ei).astype(jnp.float32)
    eye = (ej == ei).astype(jnp.float32)
    cumblocks_col = lax.dot_general(
        lincl, blocks_f, (((1,), (1,)), ((), ())),
        preferred_element_type=jnp.float32).astype(jnp.int32)
    blocks_col = lax.dot_general(
        eye, blocks_f, (((1,), (1,)), ((), ())),
        preferred_element_type=jnp.float32).astype(jnp.int32)
    counts_col = lax.dot_general(
        eye, counts, (((1,), (1,)), ((), ())),
        preferred_element_type=jnp.float32).astype(jnp.int32)
    base_col = BT * (cumblocks_col - blocks_col)
    nused = jnp.sum(blocks, axis=-1, keepdims=True)

    basesel0 = lax.dot_general(oh0, base_col.astype(jnp.float32),
                               (((1,), (0,)), ((), ())),
                               preferred_element_type=jnp.float32)
    basesel1 = lax.dot_general(oh1, base_col.astype(jnp.float32),
                               (((1,), (0,)), ((), ())),
                               preferred_element_type=jnp.float32)
    rank0 = jnp.sum(oh0 * cntb, axis=-1, keepdims=True)
    rank1 = jnp.sum(oh1 * cntb, axis=-1, keepdims=True)
    inv0_ref[...] = (basesel0 + rank0).astype(jnp.int32)
    inv1_ref[...] = (basesel1 + rank1).astype(jnp.int32)

    b_lane = lax.broadcasted_iota(jnp.int32, (NUM_EXPERTS, NBMAX), 1)
    ones_b = jnp.ones((1, NBMAX), jnp.int32)
    cum_b = cumblocks_col * ones_b
    blk_b = blocks_col * ones_b
    bexp = jnp.sum((cum_b <= b_lane).astype(jnp.int32), axis=0,
                   keepdims=True)
    bexp_ref[...] = jnp.minimum(bexp, NUM_EXPERTS - 1)
    ind = ((cum_b > b_lane) & (cum_b - blk_b <= b_lane)).astype(jnp.int32)
    gend_col = base_col + counts_col
    gend_ref[...] = jnp.sum(ind * (gend_col * ones_b), axis=0, keepdims=True)
    meta_ref[...] = nused * jnp.ones((1, 8), jnp.int32)


def _router(x, gate_w):
    return pl.pallas_call(
        _router_body,
        out_shape=[
            jax.ShapeDtypeStruct((T, 1), jnp.int32),
            jax.ShapeDtypeStruct((T, 1), jnp.int32),
            jax.ShapeDtypeStruct((T, 1), jnp.float32),
            jax.ShapeDtypeStruct((T, 1), jnp.float32),
            jax.ShapeDtypeStruct((1, NBMAX), jnp.int32),
            jax.ShapeDtypeStruct((1, NBMAX), jnp.int32),
            jax.ShapeDtypeStruct((1, 8), jnp.int32),
        ],
    )(x, gate_w)


_TOKW = T // NW


def _xs_scatter_body(x_hbm, inv0_hbm, inv1_hbm, w0_hbm, w1_hbm,
                     xs_hbm, ws_hbm, p0_v, p1_v, w0_v, w1_v, xrows,
                     gsem, s0, s1, s2, s3):
    wid = lax.axis_index("s") * NC + lax.axis_index("c")
    tbase = wid * _TOKW
    ld = pltpu.async_copy(x_hbm.at[pl.ds(tbase, _TOKW)], xrows, gsem)
    pltpu.sync_copy(inv0_hbm.at[pl.ds(tbase, _TOKW)], p0_v)
    pltpu.sync_copy(inv1_hbm.at[pl.ds(tbase, _TOKW)], p1_v)
    pltpu.sync_copy(w0_hbm.at[pl.ds(tbase, _TOKW)], w0_v)
    pltpu.sync_copy(w1_hbm.at[pl.ds(tbase, _TOKW)], w1_v)
    st2 = pltpu.async_copy(w0_v, ws_hbm.at[p0_v], s2)
    st3 = pltpu.async_copy(w1_v, ws_hbm.at[p1_v], s3)
    ld.wait()
    st0 = pltpu.async_copy(xrows, xs_hbm.at[p0_v], s0)
    st1 = pltpu.async_copy(xrows, xs_hbm.at[p1_v], s1)
    st0.wait()
    st1.wait()
    st2.wait()
    st3.wait()


@functools.cache
def _make_xs_scatter():
    return pl.kernel(
        _xs_scatter_body,
        out_type=[
            jax.ShapeDtypeStruct((SP, HIDDEN), jnp.float32),
            jax.ShapeDtypeStruct((SP,), jnp.float32),
        ],
        mesh=plsc.VectorSubcoreMesh(core_axis_name="c", subcore_axis_name="s",
                                    num_cores=NC, num_subcores=NS),
        scratch_types=[
            pltpu.VMEM((_TOKW,), jnp.int32),
            pltpu.VMEM((_TOKW,), jnp.int32),
            pltpu.VMEM((_TOKW,), jnp.float32),
            pltpu.VMEM((_TOKW,), jnp.float32),
            pltpu.VMEM((_TOKW, HIDDEN), jnp.float32),
            pltpu.SemaphoreType.DMA,
            pltpu.SemaphoreType.DMA,
            pltpu.SemaphoreType.DMA,
            pltpu.SemaphoreType.DMA,
            pltpu.SemaphoreType.DMA,
        ],
    )


def _xs_scatter(x, inv0, inv1, w0, w1):
    return _make_xs_scatter()(x, inv0, inv1, w0, w1)


def _gmm_body(bexp, meta, gend, xs_ref, w1_ref, w3_ref, w2_ref, ws_ref,
              yin_ref, ys_ref):
    f = pl.program_id(0)
    b = pl.program_id(1)
    nused = meta[0]

    @pl.when(b < nused)
    def _():
        x = xs_ref[...]
        h1 = lax.dot_general(x, w1_ref[0], (((1,), (1,)), ((), ())),
                             preferred_element_type=jnp.float32)
        h3 = lax.dot_general(x, w3_ref[0], (((1,), (1,)), ((), ())),
                             preferred_element_type=jnp.float32)
        act = h1 * (1.0 / (1.0 + jnp.exp(-h1))) * h3
        y = lax.dot_general(act, w2_ref[0], (((1,), (1,)), ((), ())),
                            preferred_element_type=jnp.float32)

        @pl.when(f == 0)
        def _():
            ys_ref[...] = y

        @pl.when((f > 0) & (f < NF - 1))
        def _():
            ys_ref[...] = yin_ref[...] + y

        @pl.when(f == NF - 1)
        def _():
            rows = b * BT + lax.broadcasted_iota(jnp.int32, (BT, 1), 0)
            wsv = jnp.where(rows < gend[b], ws_ref[...], 0.0)
            ys_ref[...] = (yin_ref[...] + y) * wsv


def _gmm(xs, w1, w3, w2, ws2d, bexp, meta, gend):
    grid_spec = pltpu.PrefetchScalarGridSpec(
        num_scalar_prefetch=3,
        grid=(NF, NBMAX),
        in_specs=[
            pl.BlockSpec((BT, HIDDEN),
                         lambda f, b, be, mt, ge:
                         (jnp.minimum(b, mt[0] - 1), 0)),
            pl.BlockSpec((1, FB, HIDDEN),
                         lambda f, b, be, mt, ge:
                         (be[jnp.minimum(b, mt[0] - 1)], f, 0)),
            pl.BlockSpec((1, FB, HIDDEN),
                         lambda f, b, be, mt, ge:
                         (be[jnp.minimum(b, mt[0] - 1)], f, 0)),
            pl.BlockSpec((1, HIDDEN, FB),
                         lambda f, b, be, mt, ge:
                         (be[jnp.minimum(b, mt[0] - 1)], 0, f)),
            pl.BlockSpec((BT, 1),
                         lambda f, b, be, mt, ge:
                         (jnp.minimum(b, mt[0] - 1), 0)),
            pl.BlockSpec((BT, HIDDEN),
                         lambda f, b, be, mt, ge:
                         (jnp.where(f == 0, NBMAX - 1,
                                    jnp.minimum(b, mt[0] - 1)), 0)),
        ],
        out_specs=pl.BlockSpec((BT, HIDDEN),
                               lambda f, b, be, mt, ge:
                               (jnp.minimum(b, mt[0] - 1), 0)),
    )
    yin = jnp.zeros((SP, HIDDEN), jnp.float32)
    return pl.pallas_call(
        _gmm_body,
        grid_spec=grid_spec,
        out_shape=jax.ShapeDtypeStruct((SP, HIDDEN), jnp.float32),
        input_output_aliases={8: 0},
        compiler_params=pltpu.CompilerParams(
            dimension_semantics=("arbitrary", "arbitrary"),
        ),
    )(bexp, meta, gend, xs, w1, w3, w2, ws2d, yin)


_TPW = T // NW
_TCH = 16
_NCC = _TPW // _TCH


def _combine_body(ys_hbm, inv0_hbm, inv1_hbm, out_hbm, p0_v, p1_v,
                  b0a, b0b, b1a, b1b, acc0, acc1,
                  g0, g1, g2, g3, s0, s1):
    wid = lax.axis_index("s") * NC + lax.axis_index("c")
    tbase = wid * _TPW
    pltpu.sync_copy(inv0_hbm.at[pl.ds(tbase, _TPW)], p0_v)
    pltpu.sync_copy(inv1_hbm.at[pl.ds(tbase, _TPW)], p1_v)
    bufs0 = (b0a, b0b)
    bufs1 = (b1a, b1b)
    accs = (acc0, acc1)
    gsem0 = (g0, g1)
    gsem1 = (g2, g3)
    ssems = (s0, s1)
    gath0 = [None] * _NCC
    gath1 = [None] * _NCC
    stores = [None] * _NCC
    for c in range(2):
        gath0[c] = pltpu.async_copy(
            ys_hbm.at[p0_v.at[pl.ds(c * _TCH, _TCH)]], bufs0[c], gsem0[c])
        gath1[c] = pltpu.async_copy(
            ys_hbm.at[p1_v.at[pl.ds(c * _TCH, _TCH)]], bufs1[c], gsem1[c])
    for c in range(_NCC):
        gath0[c].wait()
        gath1[c].wait()
        if c >= 2:
            stores[c - 2].wait()
        u = c % 2
        pa = bufs0[u]
        pb = bufs1[u]
        acc_v = accs[u]

        def add_body(i, carry):
            r = i // (HIDDEN // 64)
            q = (i % (HIDDEN // 64)) * 64
            for k in range(4):
                acc_v[r, pl.ds(q + k * 16, 16)] = (
                    pa[r, pl.ds(q + k * 16, 16)]
                    + pb[r, pl.ds(q + k * 16, 16)])
            return carry

        lax.fori_loop(0, _TCH * (HIDDEN // 64), add_body, 0)
        stores[c] = pltpu.async_copy(
            acc_v, out_hbm.at[pl.ds(tbase + c * _TCH, _TCH)], ssems[u])
        if c + 2 < _NCC:
            gath0[c + 2] = pltpu.async_copy(
                ys_hbm.at[p0_v.at[pl.ds((c + 2) * _TCH, _TCH)]],
                bufs0[u], gsem0[u])
            gath1[c + 2] = pltpu.async_copy(
                ys_hbm.at[p1_v.at[pl.ds((c + 2) * _TCH, _TCH)]],
                bufs1[u], gsem1[u])
    stores[_NCC - 2].wait()
    stores[_NCC - 1].wait()


@functools.cache
def _make_combine():
    return pl.kernel(
        _combine_body,
        out_type=jax.ShapeDtypeStruct((T, HIDDEN), jnp.float32),
        mesh=plsc.VectorSubcoreMesh(core_axis_name="c", subcore_axis_name="s",
                                    num_cores=NC, num_subcores=NS),
        scratch_types=[
            pltpu.VMEM((_TPW,), jnp.int32),
            pltpu.VMEM((_TPW,), jnp.int32),
            pltpu.VMEM((_TCH, HIDDEN), jnp.float32),
            pltpu.VMEM((_TCH, HIDDEN), jnp.float32),
            pltpu.VMEM((_TCH, HIDDEN), jnp.float32),
            pltpu.VMEM((_TCH, HIDDEN), jnp.float32),
            pltpu.VMEM((_TCH, HIDDEN), jnp.float32),
            pltpu.VMEM((_TCH, HIDDEN), jnp.float32),
            pltpu.SemaphoreType.DMA,
            pltpu.SemaphoreType.DMA,
            pltpu.SemaphoreType.DMA,
            pltpu.SemaphoreType.DMA,
            pltpu.SemaphoreType.DMA,
            pltpu.SemaphoreType.DMA,
        ],
    )


def _combine(ys, inv0, inv1):
    return _make_combine()(ys, inv0, inv1)


@jax.jit
def _moe(x, gate_w, w1, w2, w3):
    inv0, inv1, w0, wv1, bexp, gend, meta = _router(x, gate_w)
    inv0 = inv0.reshape(T)
    inv1 = inv1.reshape(T)
    xs, ws = _xs_scatter(x, inv0, inv1, w0.reshape(T), wv1.reshape(T))
    ys = _gmm(xs, w1, w3, w2, ws.reshape(SP, 1),
              bexp.reshape(NBMAX), meta.reshape(8), gend.reshape(NBMAX))
    return _combine(ys, inv0, inv1)


def kernel(hidden_states, gate_w, w1, w2, w3):
    B, S, H = hidden_states.shape
    x = hidden_states.reshape(-1, H)
    out = _moe(x, gate_w, w1, w2, w3)
    return out.reshape(B, S, H)
